# baseline (speedup 1.0000x reference)
import functools

import jax
import jax.numpy as jnp
from jax.experimental import pallas as pl
from jax.experimental.pallas import tpu as pltpu

N = 10000
E = 320000
BM = 2000


def _rdeg(deg_blk):
    return 1.0 / jnp.maximum(deg_blk, 1.0)


def _t1_kernel(x_ref, p0_ref, p1_ref, deg_ref, wl_ref, wr_ref, b_ref, o_ref):
    agg = (p0_ref[...] + p1_ref[...]) * _rdeg(deg_ref[...])
    acc = jnp.dot(agg, wl_ref[...], preferred_element_type=jnp.float32)
    acc += jnp.dot(x_ref[...], wr_ref[...], preferred_element_type=jnp.float32)
    acc += b_ref[...]
    o_ref[...] = jnp.maximum(acc, 0.0).astype(jnp.bfloat16)


def _t1(x, p0, p1, deg, Wl, Wr, b):
    d_in, d_out = Wl.shape
    bn = 512
    return pl.pallas_call(
        _t1_kernel,
        grid=(N // BM, d_out // bn),
        in_specs=[
            pl.BlockSpec((BM, d_in), lambda i, j: (i, 0)),
            pl.BlockSpec((BM, d_in), lambda i, j: (i, 0)),
            pl.BlockSpec((BM, d_in), lambda i, j: (i, 0)),
            pl.BlockSpec((BM, 1), lambda i, j: (i, 0)),
            pl.BlockSpec((d_in, bn), lambda i, j: (0, j)),
            pl.BlockSpec((d_in, bn), lambda i, j: (0, j)),
            pl.BlockSpec((1, bn), lambda i, j: (0, j)),
        ],
        out_specs=pl.BlockSpec((BM, bn), lambda i, j: (i, j)),
        out_shape=jax.ShapeDtypeStruct((N, d_out), jnp.bfloat16),
    )(x, p0, p1, deg, Wl, Wr, b.reshape(1, d_out))


def _t2a_kernel(h_ref, w_ref, o_ref):
    o_ref[0] = jnp.dot(h_ref[...], w_ref[...], preferred_element_type=jnp.float32)


def _t2a(h1, Wl):
    k, d_out = Wl.shape
    nc = d_out // 128
    return pl.pallas_call(
        _t2a_kernel,
        grid=(N // BM, nc),
        in_specs=[
            pl.BlockSpec((BM, k), lambda i, j: (i, 0)),
            pl.BlockSpec((k, 128), lambda i, j: (0, j)),
        ],
        out_specs=pl.BlockSpec((1, BM, 128), lambda i, j: (j, i, 0)),
        out_shape=jax.ShapeDtypeStruct((nc, N, 128), jnp.float32),
    )(h1, Wl)


def _t2b_kernel(h_ref, agg_ref, deg_ref, w_ref, b_ref, o_ref):
    acc = jnp.dot(h_ref[...], w_ref[...], preferred_element_type=jnp.float32)
    acc += agg_ref[0] * _rdeg(deg_ref[...])
    acc += b_ref[...]
    o_ref[...] = jnp.maximum(acc, 0.0).astype(jnp.bfloat16)


def _t2b(h1, agg2, deg, Wr, b):
    k, d_out = Wr.shape
    nc = d_out // 128
    return pl.pallas_call(
        _t2b_kernel,
        grid=(N // BM, nc),
        in_specs=[
            pl.BlockSpec((BM, k), lambda i, j: (i, 0)),
            pl.BlockSpec((1, BM, 128), lambda i, j: (j, i, 0)),
            pl.BlockSpec((BM, 1), lambda i, j: (i, 0)),
            pl.BlockSpec((k, 128), lambda i, j: (0, j)),
            pl.BlockSpec((1, 128), lambda i, j: (0, j)),
        ],
        out_specs=pl.BlockSpec((BM, 128), lambda i, j: (i, j)),
        out_shape=jax.ShapeDtypeStruct((N, d_out), jnp.bfloat16),
    )(h1, agg2, deg, Wr, b.reshape(1, d_out))


def _t3a_kernel(h_ref, w_ref, o_ref):
    h = h_ref[...].astype(jnp.float32)
    o_ref[...] = jnp.dot(h, w_ref[...], preferred_element_type=jnp.float32)


def _t3a(h2, Wl):
    k, d_out = Wl.shape
    return pl.pallas_call(
        _t3a_kernel,
        grid=(N // BM,),
        in_specs=[
            pl.BlockSpec((BM, k), lambda i: (i, 0)),
            pl.BlockSpec((k, d_out), lambda i: (0, 0)),
        ],
        out_specs=pl.BlockSpec((BM, d_out), lambda i: (i, 0)),
        out_shape=jax.ShapeDtypeStruct((N, d_out), jnp.float32),
    )(h2, Wl)


def _t3b_kernel(h_ref, p0_ref, p1_ref, deg_ref, w_ref, b_ref, o_ref):
    h = h_ref[...].astype(jnp.float32)
    acc = jnp.dot(h, w_ref[...], preferred_element_type=jnp.float32)
    acc += (p0_ref[...] + p1_ref[...]) * _rdeg(deg_ref[...])
    acc += b_ref[...]
    m = jnp.max(acc, axis=1, keepdims=True)
    s = acc - m
    lse = jnp.log(jnp.sum(jnp.exp(s), axis=1, keepdims=True))
    o_ref[...] = s - lse


def _t3b(h2, p0, p1, deg, Wr, b):
    k, d_out = Wr.shape
    return pl.pallas_call(
        _t3b_kernel,
        grid=(N // BM,),
        in_specs=[
            pl.BlockSpec((BM, k), lambda i: (i, 0)),
            pl.BlockSpec((BM, d_out), lambda i: (i, 0)),
            pl.BlockSpec((BM, d_out), lambda i: (i, 0)),
            pl.BlockSpec((BM, 1), lambda i: (i, 0)),
            pl.BlockSpec((k, d_out), lambda i: (0, 0)),
            pl.BlockSpec((1, d_out), lambda i: (0, 0)),
        ],
        out_specs=pl.BlockSpec((BM, d_out), lambda i: (i, 0)),
        out_shape=jax.ShapeDtypeStruct((N, d_out), jnp.float32),
    )(h2, p0, p1, deg, Wr, b.reshape(1, d_out))


def kernel(x, edge_index, W1l, W1r, b1, W2l, W2r, b2, W3l, W3r, b3):
    src = edge_index[0]
    dst = edge_index[1]
    deg = jax.ops.segment_sum(jnp.ones((E,), jnp.float32), dst, num_segments=N)
    deg = deg.reshape(N, 1)
    zeros128 = jnp.zeros((N, 128), jnp.float32)
    zeros64 = jnp.zeros((N, 64), jnp.float32)

    p0 = jax.ops.segment_sum(x[src], dst, num_segments=N)
    h1 = _t1(x, p0, zeros128, deg, W1l, W1r, b1)

    m2 = _t2a(h1, W2l.astype(jnp.bfloat16))
    m2_flat = m2.transpose(1, 0, 2).reshape(N, 1024)
    agg2 = jax.ops.segment_sum(m2_flat[src], dst, num_segments=N)
    agg2 = agg2.reshape(N, 8, 128).transpose(1, 0, 2)
    h2 = _t2b(h1, agg2, deg, W2r.astype(jnp.bfloat16), b2)

    m3 = _t3a(h2, W3l)
    p3 = jax.ops.segment_sum(m3[src], dst, num_segments=N)
    return _t3b(h2, p3, zeros64, deg, W3r, b3)


# speedup vs baseline: 3.3899x; 3.3899x over previous
import functools

import jax
import jax.numpy as jnp
from jax import lax
from jax.experimental import pallas as pl
from jax.experimental.pallas import tpu as pltpu
from jax.experimental.pallas import tpu_sc as plsc

N = 10000
E = 320000
BM = 2000
B = 128

_F32 = jnp.float32

HALF_NB, HALF_EXTRA = 78, 2
FULL_NB, FULL_EXTRA = 156, 4


def _fill_zero_2d(ref, rows, cols):
    def body(r, _):
        for k in range(cols // 16):
            ref[r, pl.ds(k * 16, 16)] = jnp.zeros((16,), _F32)
        return 0
    lax.fori_loop(0, rows, body, 0)


def _fill_const_1d(ref, n, val):
    def body(k, _):
        ref[pl.ds(k * 16, 16)] = jnp.full((16,), val, _F32)
        return 0
    lax.fori_loop(0, n // 16, body, 0)


def _zero_acc_region(acc, zrow_v, s, d):
    @pl.when(s < 10)
    def _():
        row0 = s * 1000
        for k in range(25):
            pltpu.sync_copy(zrow_v, acc.at[pl.ds(row0 + k * 40, 40)])


def _load_slab(idx_hbm, idx2d, e_base, s, nb_main, extra, extra_base):
    def body(b, _):
        pltpu.sync_copy(idx_hbm.at[pl.ds(e_base + b * B, B)], idx2d.at[b])
        return 0
    lax.fori_loop(0, nb_main, body, 0)

    @pl.when(s < extra)
    def _():
        pltpu.sync_copy(idx_hbm.at[pl.ds(extra_base + s * B, B)],
                        idx2d.at[nb_main])


def _s1_body(x_hbm, src_hbm, dst_hbm, p_out, d_out,
             acc, dacc, src2d, dst2d, rows_v, ones_v, zrow_v, zdeg_v,
             dstage_v):
    c = lax.axis_index("c")
    s = lax.axis_index("s")

    _fill_zero_2d(zrow_v, 40, 128)
    _fill_const_1d(zdeg_v, 1024, 0.0)
    _fill_const_1d(ones_v, B, 1.0)

    e_base = c * (E // 2) + s * HALF_NB * B
    extra_base = c * (E // 2) + 16 * HALF_NB * B
    _load_slab(src_hbm, src2d, e_base, s, HALF_NB, HALF_EXTRA, extra_base)
    _load_slab(dst_hbm, dst2d, e_base, s, HALF_NB, HALF_EXTRA, extra_base)

    _zero_acc_region(acc, zrow_v, s, 128)

    @pl.when(s < 10)
    def _():
        pltpu.sync_copy(zdeg_v.at[pl.ds(0, 1000)], dacc.at[pl.ds(s * 1000, 1000)])

    plsc.subcore_barrier()

    nb = jnp.where(s < HALF_EXTRA, HALF_NB + 1, HALF_NB)

    def batch(b, _):
        pltpu.sync_copy(x_hbm.at[src2d.at[b]], rows_v)
        pltpu.sync_copy(rows_v, acc.at[dst2d.at[b]], add=True)
        pltpu.sync_copy(ones_v, dacc.at[dst2d.at[b]], add=True)
        return 0
    lax.fori_loop(0, nb, batch, 0)

    plsc.subcore_barrier()

    @pl.when(s < 10)
    def _():
        pltpu.sync_copy(acc.at[pl.ds(s * 1000, 1000)],
                        p_out.at[pl.ds(c * N + s * 1000, 1000)])

    @pl.when(s < 10)
    def _():
        pltpu.sync_copy(dacc.at[pl.ds(s * 1000, 1000)], dstage_v)
        pltpu.sync_copy(dstage_v, d_out.at[pl.ds(c * N + s * 1000, 1000)])


def _make_s1():
  return pl.kernel(
    _s1_body,
    out_type=[jax.ShapeDtypeStruct((2 * N, 128), _F32),
              jax.ShapeDtypeStruct((2 * N,), _F32)],
    mesh=plsc.VectorSubcoreMesh(core_axis_name="c", subcore_axis_name="s"),
    scratch_types=[
        pltpu.VMEM_SHARED((N, 128), _F32),
        pltpu.VMEM_SHARED((N,), _F32),
        pltpu.VMEM((HALF_NB + 1, B), jnp.int32),
        pltpu.VMEM((HALF_NB + 1, B), jnp.int32),
        pltpu.VMEM((B, 128), _F32),
        pltpu.VMEM((B,), _F32),
        pltpu.VMEM((40, 128), _F32),
        pltpu.VMEM((1024,), _F32),
        pltpu.VMEM((1000,), _F32),
    ],
  )


def _s2_body(m2_hbm, src_hbm, dst_hbm, agg_out,
             acc, dst2d, src_v, adj_v, rows_v, zrow_v):
    c = lax.axis_index("c")
    s = lax.axis_index("s")

    _fill_zero_2d(zrow_v, 40, 128)

    e_base = s * FULL_NB * B
    extra_base = 16 * FULL_NB * B
    _load_slab(dst_hbm, dst2d, e_base, s, FULL_NB, FULL_EXTRA, extra_base)

    nb = jnp.where(s < FULL_EXTRA, FULL_NB + 1, FULL_NB)

    for cc_local in range(4):
        cc = c * 4 + cc_local
        base = cc * N

        _zero_acc_region(acc, zrow_v, s, 128)
        plsc.subcore_barrier()

        def batch(b, _):
            e0 = jnp.where(b == FULL_NB, extra_base + s * B, e_base + b * B)
            pltpu.sync_copy(src_hbm.at[pl.ds(e0, B)], src_v)
            for k in range(B // 16):
                adj_v[pl.ds(k * 16, 16)] = src_v[pl.ds(k * 16, 16)] + base
            pltpu.sync_copy(m2_hbm.at[adj_v], rows_v)
            pltpu.sync_copy(rows_v, acc.at[dst2d.at[b]], add=True)
            return 0
        lax.fori_loop(0, nb, batch, 0)

        plsc.subcore_barrier()

        @pl.when(s < 10)
        def _(base=base):
            pltpu.sync_copy(acc.at[pl.ds(s * 1000, 1000)],
                            agg_out.at[pl.ds(base + s * 1000, 1000)])
        plsc.subcore_barrier()


def _make_s2():
  return pl.kernel(
    _s2_body,
    out_type=jax.ShapeDtypeStruct((8 * N, 128), _F32),
    mesh=plsc.VectorSubcoreMesh(core_axis_name="c", subcore_axis_name="s"),
    scratch_types=[
        pltpu.VMEM_SHARED((N, 128), _F32),
        pltpu.VMEM((FULL_NB + 1, B), jnp.int32),
        pltpu.VMEM((B,), jnp.int32),
        pltpu.VMEM((B,), jnp.int32),
        pltpu.VMEM((B, 128), _F32),
        pltpu.VMEM((40, 128), _F32),
    ],
  )


def _s3_body(m3_hbm, src_hbm, dst_hbm, p_out,
             acc, src2d, dst2d, rows_v, zrow_v):
    c = lax.axis_index("c")
    s = lax.axis_index("s")

    _fill_zero_2d(zrow_v, 40, 128)

    e_base = c * (E // 2) + s * HALF_NB * B
    extra_base = c * (E // 2) + 16 * HALF_NB * B
    _load_slab(src_hbm, src2d, e_base, s, HALF_NB, HALF_EXTRA, extra_base)
    _load_slab(dst_hbm, dst2d, e_base, s, HALF_NB, HALF_EXTRA, extra_base)

    _zero_acc_region(acc, zrow_v, s, 128)
    plsc.subcore_barrier()

    nb = jnp.where(s < HALF_EXTRA, HALF_NB + 1, HALF_NB)

    def batch(b, _):
        pltpu.sync_copy(m3_hbm.at[src2d.at[b]], rows_v)
        pltpu.sync_copy(rows_v, acc.at[dst2d.at[b]], add=True)
        return 0
    lax.fori_loop(0, nb, batch, 0)

    plsc.subcore_barrier()

    @pl.when(s < 10)
    def _():
        pltpu.sync_copy(acc.at[pl.ds(s * 1000, 1000)],
                        p_out.at[pl.ds(c * N + s * 1000, 1000)])


def _make_s3():
  return pl.kernel(
    _s3_body,
    out_type=jax.ShapeDtypeStruct((2 * N, 128), _F32),
    mesh=plsc.VectorSubcoreMesh(core_axis_name="c", subcore_axis_name="s"),
    scratch_types=[
        pltpu.VMEM_SHARED((N, 128), _F32),
        pltpu.VMEM((HALF_NB + 1, B), jnp.int32),
        pltpu.VMEM((HALF_NB + 1, B), jnp.int32),
        pltpu.VMEM((B, 128), _F32),
        pltpu.VMEM((40, 128), _F32),
    ],
  )


def _rdeg(d0, d1):
    return 1.0 / jnp.maximum(d0 + d1, 1.0)


def _t1_kernel(x_ref, p0_ref, p1_ref, d0_ref, d1_ref, wl_ref, wr_ref, b_ref,
               o_ref):
    agg = (p0_ref[0] + p1_ref[0]) * _rdeg(d0_ref[0], d1_ref[0])
    acc = jnp.dot(agg, wl_ref[...], preferred_element_type=_F32)
    acc += jnp.dot(x_ref[...], wr_ref[...], preferred_element_type=_F32)
    acc += b_ref[...]
    o_ref[...] = jnp.maximum(acc, 0.0).astype(jnp.bfloat16)


def _t1(x, pp, dd, Wl, Wr, b):
    d_in, d_out = Wl.shape
    bn = 512
    return pl.pallas_call(
        _t1_kernel,
        grid=(N // BM, d_out // bn),
        in_specs=[
            pl.BlockSpec((BM, d_in), lambda i, j: (i, 0)),
            pl.BlockSpec((1, BM, d_in), lambda i, j: (0, i, 0)),
            pl.BlockSpec((1, BM, d_in), lambda i, j: (1, i, 0)),
            pl.BlockSpec((1, BM, 1), lambda i, j: (0, i, 0)),
            pl.BlockSpec((1, BM, 1), lambda i, j: (1, i, 0)),
            pl.BlockSpec((d_in, bn), lambda i, j: (0, j)),
            pl.BlockSpec((d_in, bn), lambda i, j: (0, j)),
            pl.BlockSpec((1, bn), lambda i, j: (0, j)),
        ],
        out_specs=pl.BlockSpec((BM, bn), lambda i, j: (i, j)),
        out_shape=jax.ShapeDtypeStruct((N, d_out), jnp.bfloat16),
    )(x, pp, pp, dd, dd, Wl, Wr, b.reshape(1, d_out))


def _t2a_kernel(h_ref, w_ref, o_ref):
    o_ref[0] = jnp.dot(h_ref[...], w_ref[...], preferred_element_type=_F32)


def _t2a(h1, Wl):
    k, d_out = Wl.shape
    nc = d_out // 128
    return pl.pallas_call(
        _t2a_kernel,
        grid=(N // BM, nc),
        in_specs=[
            pl.BlockSpec((BM, k), lambda i, j: (i, 0)),
            pl.BlockSpec((k, 128), lambda i, j: (0, j)),
        ],
        out_specs=pl.BlockSpec((1, BM, 128), lambda i, j: (j, i, 0)),
        out_shape=jax.ShapeDtypeStruct((nc, N, 128), _F32),
    )(h1, Wl)


def _t2b_kernel(h_ref, agg_ref, d0_ref, d1_ref, w_ref, b_ref, o_ref):
    acc = jnp.dot(h_ref[...], w_ref[...], preferred_element_type=_F32)
    acc += agg_ref[0] * _rdeg(d0_ref[0], d1_ref[0])
    acc += b_ref[...]
    o_ref[...] = jnp.maximum(acc, 0.0).astype(jnp.bfloat16)


def _t2b(h1, agg2, dd, Wr, b):
    k, d_out = Wr.shape
    nc = d_out // 128
    return pl.pallas_call(
        _t2b_kernel,
        grid=(N // BM, nc),
        in_specs=[
            pl.BlockSpec((BM, k), lambda i, j: (i, 0)),
            pl.BlockSpec((1, BM, 128), lambda i, j: (j, i, 0)),
            pl.BlockSpec((1, BM, 1), lambda i, j: (0, i, 0)),
            pl.BlockSpec((1, BM, 1), lambda i, j: (1, i, 0)),
            pl.BlockSpec((k, 128), lambda i, j: (0, j)),
            pl.BlockSpec((1, 128), lambda i, j: (0, j)),
        ],
        out_specs=pl.BlockSpec((BM, 128), lambda i, j: (i, j)),
        out_shape=jax.ShapeDtypeStruct((N, d_out), jnp.bfloat16),
    )(h1, agg2, dd, dd, Wr, b.reshape(1, d_out))


def _t3a_kernel(h_ref, w_ref, o_ref):
    h = h_ref[...].astype(_F32)
    o_ref[...] = jnp.dot(h, w_ref[...], preferred_element_type=_F32)


def _t3a(h2, Wl):
    k, d_out = Wl.shape
    return pl.pallas_call(
        _t3a_kernel,
        grid=(N // BM,),
        in_specs=[
            pl.BlockSpec((BM, k), lambda i: (i, 0)),
            pl.BlockSpec((k, d_out), lambda i: (0, 0)),
        ],
        out_specs=pl.BlockSpec((BM, d_out), lambda i: (i, 0)),
        out_shape=jax.ShapeDtypeStruct((N, d_out), _F32),
    )(h2, Wl)


def _t3b_kernel(h_ref, p0_ref, p1_ref, d0_ref, d1_ref, w_ref, b_ref, o_ref):
    h = h_ref[...].astype(_F32)
    acc = jnp.dot(h, w_ref[...], preferred_element_type=_F32)
    acc += (p0_ref[0][:, :64] + p1_ref[0][:, :64]) * _rdeg(d0_ref[0], d1_ref[0])
    acc += b_ref[...]
    m = jnp.max(acc, axis=1, keepdims=True)
    sh = acc - m
    lse = jnp.log(jnp.sum(jnp.exp(sh), axis=1, keepdims=True))
    o_ref[...] = sh - lse


def _t3b(h2, pp, dd, Wr, b):
    k, d_out = Wr.shape
    return pl.pallas_call(
        _t3b_kernel,
        grid=(N // BM,),
        in_specs=[
            pl.BlockSpec((BM, k), lambda i: (i, 0)),
            pl.BlockSpec((1, BM, 128), lambda i: (0, i, 0)),
            pl.BlockSpec((1, BM, 128), lambda i: (1, i, 0)),
            pl.BlockSpec((1, BM, 1), lambda i: (0, i, 0)),
            pl.BlockSpec((1, BM, 1), lambda i: (1, i, 0)),
            pl.BlockSpec((k, d_out), lambda i: (0, 0)),
            pl.BlockSpec((1, d_out), lambda i: (0, 0)),
        ],
        out_specs=pl.BlockSpec((BM, d_out), lambda i: (i, 0)),
        out_shape=jax.ShapeDtypeStruct((N, d_out), _F32),
    )(h2, pp, pp, dd, dd, Wr, b.reshape(1, d_out))


@functools.lru_cache(maxsize=1)
def _sc_kernels():
    return _make_s1(), _make_s2(), _make_s3()


def kernel(x, edge_index, W1l, W1r, b1, W2l, W2r, b2, W3l, W3r, b3):
    src = edge_index[0]
    dst = edge_index[1]
    _s1, _s2, _s3 = _sc_kernels()

    pp, ddflat = _s1(x, src, dst)
    pp = pp.reshape(2, N, 128)
    dd = ddflat.reshape(2, N, 1)

    h1 = _t1(x, pp, dd, W1l, W1r, b1)

    m2 = _t2a(h1, W2l.astype(jnp.bfloat16))
    agg2 = _s2(m2.reshape(8 * N, 128), src, dst).reshape(8, N, 128)
    h2 = _t2b(h1, agg2, dd, W2r.astype(jnp.bfloat16), b2)

    W3lp = jnp.concatenate([W3l, jnp.zeros((W3l.shape[0], 64), _F32)], axis=1)
    m3 = _t3a(h2, W3lp)
    p3 = _s3(m3, src, dst).reshape(2, N, 128)
    return _t3b(h2, p3, dd, W3r, b3)


# speedup vs baseline: 4.0129x; 1.1838x over previous
import functools

import jax
import jax.numpy as jnp
from jax import lax
from jax.experimental import pallas as pl
from jax.experimental.pallas import tpu as pltpu
from jax.experimental.pallas import tpu_sc as plsc

N = 10000
E = 320000
BM = 2000
B = 128

_F32 = jnp.float32

HALF_NB, HALF_EXTRA = 78, 2
FULL_NB, FULL_EXTRA = 156, 4


def _fill_zero_2d(ref, rows, cols):
    def body(r, _):
        for k in range(cols // 16):
            ref[r, pl.ds(k * 16, 16)] = jnp.zeros((16,), _F32)
        return 0
    lax.fori_loop(0, rows, body, 0)


def _fill_const_1d(ref, n, val):
    def body(k, _):
        ref[pl.ds(k * 16, 16)] = jnp.full((16,), val, _F32)
        return 0
    lax.fori_loop(0, n // 16, body, 0)


def _zero_acc_region(acc, zrow_v, s, d):
    @pl.when(s < 10)
    def _():
        row0 = s * 1000
        for k in range(25):
            pltpu.sync_copy(zrow_v, acc.at[pl.ds(row0 + k * 40, 40)])


def _load_slab(idx_hbm, idx2d, e_base, s, nb_main, extra, extra_base):
    def body(b, _):
        pltpu.sync_copy(idx_hbm.at[pl.ds(e_base + b * B, B)], idx2d.at[b])
        return 0
    lax.fori_loop(0, nb_main, body, 0)

    @pl.when(s < extra)
    def _():
        pltpu.sync_copy(idx_hbm.at[pl.ds(extra_base + s * B, B)],
                        idx2d.at[nb_main])


def _s1_body(x_hbm, src_hbm, dst_hbm, p_out, d_out,
             acc, dacc, src2d, dst2d, rows_v, ones_v, zrow_v, zdeg_v,
             dstage_v):
    c = lax.axis_index("c")
    s = lax.axis_index("s")

    _fill_zero_2d(zrow_v, 40, 128)
    _fill_const_1d(zdeg_v, 1024, 0.0)
    _fill_const_1d(ones_v, B, 1.0)

    e_base = c * (E // 2) + s * HALF_NB * B
    extra_base = c * (E // 2) + 16 * HALF_NB * B
    _load_slab(src_hbm, src2d, e_base, s, HALF_NB, HALF_EXTRA, extra_base)
    _load_slab(dst_hbm, dst2d, e_base, s, HALF_NB, HALF_EXTRA, extra_base)

    _zero_acc_region(acc, zrow_v, s, 128)

    @pl.when(s < 10)
    def _():
        pltpu.sync_copy(zdeg_v.at[pl.ds(0, 1000)], dacc.at[pl.ds(s * 1000, 1000)])

    plsc.subcore_barrier()

    nb = jnp.where(s < HALF_EXTRA, HALF_NB + 1, HALF_NB)

    def batch(b, _):
        pltpu.sync_copy(x_hbm.at[src2d.at[b]], rows_v)
        pltpu.sync_copy(rows_v, acc.at[dst2d.at[b]], add=True)
        pltpu.sync_copy(ones_v, dacc.at[dst2d.at[b]], add=True)
        return 0
    lax.fori_loop(0, nb, batch, 0)

    plsc.subcore_barrier()

    @pl.when(s < 10)
    def _():
        pltpu.sync_copy(acc.at[pl.ds(s * 1000, 1000)],
                        p_out.at[pl.ds(c * N + s * 1000, 1000)])

    @pl.when(s < 10)
    def _():
        pltpu.sync_copy(dacc.at[pl.ds(s * 1000, 1000)], dstage_v)
        pltpu.sync_copy(dstage_v, d_out.at[pl.ds(c * N + s * 1000, 1000)])


def _make_s1():
  return pl.kernel(
    _s1_body,
    out_type=[jax.ShapeDtypeStruct((2 * N, 128), _F32),
              jax.ShapeDtypeStruct((2 * N,), _F32)],
    mesh=plsc.VectorSubcoreMesh(core_axis_name="c", subcore_axis_name="s"),
    scratch_types=[
        pltpu.VMEM_SHARED((N, 128), _F32),
        pltpu.VMEM_SHARED((N,), _F32),
        pltpu.VMEM((HALF_NB + 1, B), jnp.int32),
        pltpu.VMEM((HALF_NB + 1, B), jnp.int32),
        pltpu.VMEM((B, 128), _F32),
        pltpu.VMEM((B,), _F32),
        pltpu.VMEM((40, 128), _F32),
        pltpu.VMEM((1024,), _F32),
        pltpu.VMEM((1000,), _F32),
    ],
  )


def _s2_body(m2_hbm, src_hbm, dst_hbm, agg_out,
             acc, sA, sB, d2, rA, rB, zrow_v, isA, idA, isB, idB, gA, gB):
    c = lax.axis_index("c")
    s = lax.axis_index("s")

    _fill_zero_2d(zrow_v, 40, 128)

    e_base = s * FULL_NB * B
    extra_base = 16 * FULL_NB * B

    for cc_local in range(4):
        cc = c * 4 + cc_local
        base = cc * N

        _zero_acc_region(acc, zrow_v, s, 128)
        plsc.subcore_barrier()

        def pair(p, _):
            e0 = e_base + (2 * p) * B
            e1 = e_base + (2 * p + 1) * B
            hsa = pltpu.async_copy(src_hbm.at[pl.ds(e0, B)], sA, isA)
            hda = pltpu.async_copy(dst_hbm.at[pl.ds(e0, B)], d2.at[0], idA)
            hsb = pltpu.async_copy(src_hbm.at[pl.ds(e1, B)], sB, isB)
            hdb = pltpu.async_copy(dst_hbm.at[pl.ds(e1, B)], d2.at[1], idB)
            hsa.wait()
            for k in range(B // 16):
                sA[pl.ds(k * 16, 16)] = sA[pl.ds(k * 16, 16)] + base
            ga = pltpu.async_copy(m2_hbm.at[sA], rA, gA)
            hsb.wait()
            for k in range(B // 16):
                sB[pl.ds(k * 16, 16)] = sB[pl.ds(k * 16, 16)] + base
            gb = pltpu.async_copy(m2_hbm.at[sB], rB, gB)
            ga.wait()
            hda.wait()
            pltpu.sync_copy(rA, acc.at[d2.at[0]], add=True)
            gb.wait()
            hdb.wait()
            pltpu.sync_copy(rB, acc.at[d2.at[1]], add=True)
            return 0
        lax.fori_loop(0, FULL_NB // 2, pair, 0)

        @pl.when(s < FULL_EXTRA)
        def _(base=base):
            e0 = extra_base + s * B
            pltpu.sync_copy(src_hbm.at[pl.ds(e0, B)], sA)
            pltpu.sync_copy(dst_hbm.at[pl.ds(e0, B)], d2.at[0])
            for k in range(B // 16):
                sA[pl.ds(k * 16, 16)] = sA[pl.ds(k * 16, 16)] + base
            pltpu.sync_copy(m2_hbm.at[sA], rA)
            pltpu.sync_copy(rA, acc.at[d2.at[0]], add=True)

        plsc.subcore_barrier()

        @pl.when(s < 10)
        def _(base=base):
            pltpu.sync_copy(acc.at[pl.ds(s * 1000, 1000)],
                            agg_out.at[pl.ds(base + s * 1000, 1000)])
        plsc.subcore_barrier()


def _make_s2():
  return pl.kernel(
    _s2_body,
    out_type=jax.ShapeDtypeStruct((8 * N, 128), _F32),
    mesh=plsc.VectorSubcoreMesh(core_axis_name="c", subcore_axis_name="s"),
    scratch_types=[
        pltpu.VMEM_SHARED((N, 128), _F32),
        pltpu.VMEM((B,), jnp.int32),
        pltpu.VMEM((B,), jnp.int32),
        pltpu.VMEM((2, B), jnp.int32),
        pltpu.VMEM((B, 128), _F32),
        pltpu.VMEM((B, 128), _F32),
        pltpu.VMEM((40, 128), _F32),
        pltpu.SemaphoreType.DMA,
        pltpu.SemaphoreType.DMA,
        pltpu.SemaphoreType.DMA,
        pltpu.SemaphoreType.DMA,
        pltpu.SemaphoreType.DMA,
        pltpu.SemaphoreType.DMA,
    ],
  )


def _s3_body(m3_hbm, src_hbm, dst_hbm, p_out,
             acc, src2d, dst2d, rows_v, zrow_v):
    c = lax.axis_index("c")
    s = lax.axis_index("s")

    _fill_zero_2d(zrow_v, 40, 128)

    e_base = c * (E // 2) + s * HALF_NB * B
    extra_base = c * (E // 2) + 16 * HALF_NB * B
    _load_slab(src_hbm, src2d, e_base, s, HALF_NB, HALF_EXTRA, extra_base)
    _load_slab(dst_hbm, dst2d, e_base, s, HALF_NB, HALF_EXTRA, extra_base)

    _zero_acc_region(acc, zrow_v, s, 128)
    plsc.subcore_barrier()

    nb = jnp.where(s < HALF_EXTRA, HALF_NB + 1, HALF_NB)

    def batch(b, _):
        pltpu.sync_copy(m3_hbm.at[src2d.at[b]], rows_v)
        pltpu.sync_copy(rows_v, acc.at[dst2d.at[b]], add=True)
        return 0
    lax.fori_loop(0, nb, batch, 0)

    plsc.subcore_barrier()

    @pl.when(s < 10)
    def _():
        pltpu.sync_copy(acc.at[pl.ds(s * 1000, 1000)],
                        p_out.at[pl.ds(c * N + s * 1000, 1000)])


def _make_s3():
  return pl.kernel(
    _s3_body,
    out_type=jax.ShapeDtypeStruct((2 * N, 128), _F32),
    mesh=plsc.VectorSubcoreMesh(core_axis_name="c", subcore_axis_name="s"),
    scratch_types=[
        pltpu.VMEM_SHARED((N, 128), _F32),
        pltpu.VMEM((HALF_NB + 1, B), jnp.int32),
        pltpu.VMEM((HALF_NB + 1, B), jnp.int32),
        pltpu.VMEM((B, 128), _F32),
        pltpu.VMEM((40, 128), _F32),
    ],
  )


def _rdeg(d0, d1):
    return 1.0 / jnp.maximum(d0 + d1, 1.0)


def _t1_kernel(x_ref, p0_ref, p1_ref, d0_ref, d1_ref, wl_ref, wr_ref, b_ref,
               o_ref):
    agg = (p0_ref[0] + p1_ref[0]) * _rdeg(d0_ref[0], d1_ref[0])
    acc = jnp.dot(agg, wl_ref[...], preferred_element_type=_F32)
    acc += jnp.dot(x_ref[...], wr_ref[...], preferred_element_type=_F32)
    acc += b_ref[...]
    o_ref[...] = jnp.maximum(acc, 0.0).astype(jnp.bfloat16)


def _t1(x, pp, dd, Wl, Wr, b):
    d_in, d_out = Wl.shape
    bn = 512
    return pl.pallas_call(
        _t1_kernel,
        grid=(N // BM, d_out // bn),
        in_specs=[
            pl.BlockSpec((BM, d_in), lambda i, j: (i, 0)),
            pl.BlockSpec((1, BM, d_in), lambda i, j: (0, i, 0)),
            pl.BlockSpec((1, BM, d_in), lambda i, j: (1, i, 0)),
            pl.BlockSpec((1, BM, 1), lambda i, j: (0, i, 0)),
            pl.BlockSpec((1, BM, 1), lambda i, j: (1, i, 0)),
            pl.BlockSpec((d_in, bn), lambda i, j: (0, j)),
            pl.BlockSpec((d_in, bn), lambda i, j: (0, j)),
            pl.BlockSpec((1, bn), lambda i, j: (0, j)),
        ],
        out_specs=pl.BlockSpec((BM, bn), lambda i, j: (i, j)),
        out_shape=jax.ShapeDtypeStruct((N, d_out), jnp.bfloat16),
    )(x, pp, pp, dd, dd, Wl, Wr, b.reshape(1, d_out))


def _t2a_kernel(h_ref, w_ref, o_ref):
    o_ref[0] = jnp.dot(h_ref[...], w_ref[...], preferred_element_type=_F32)


def _t2a(h1, Wl):
    k, d_out = Wl.shape
    nc = d_out // 128
    return pl.pallas_call(
        _t2a_kernel,
        grid=(N // BM, nc),
        in_specs=[
            pl.BlockSpec((BM, k), lambda i, j: (i, 0)),
            pl.BlockSpec((k, 128), lambda i, j: (0, j)),
        ],
        out_specs=pl.BlockSpec((1, BM, 128), lambda i, j: (j, i, 0)),
        out_shape=jax.ShapeDtypeStruct((nc, N, 128), _F32),
    )(h1, Wl)


def _t2b_kernel(h_ref, agg_ref, d0_ref, d1_ref, w_ref, b_ref, o_ref):
    acc = jnp.dot(h_ref[...], w_ref[...], preferred_element_type=_F32)
    acc += agg_ref[0] * _rdeg(d0_ref[0], d1_ref[0])
    acc += b_ref[...]
    o_ref[...] = jnp.maximum(acc, 0.0).astype(jnp.bfloat16)


def _t2b(h1, agg2, dd, Wr, b):
    k, d_out = Wr.shape
    nc = d_out // 128
    return pl.pallas_call(
        _t2b_kernel,
        grid=(N // BM, nc),
        in_specs=[
            pl.BlockSpec((BM, k), lambda i, j: (i, 0)),
            pl.BlockSpec((1, BM, 128), lambda i, j: (j, i, 0)),
            pl.BlockSpec((1, BM, 1), lambda i, j: (0, i, 0)),
            pl.BlockSpec((1, BM, 1), lambda i, j: (1, i, 0)),
            pl.BlockSpec((k, 128), lambda i, j: (0, j)),
            pl.BlockSpec((1, 128), lambda i, j: (0, j)),
        ],
        out_specs=pl.BlockSpec((BM, 128), lambda i, j: (i, j)),
        out_shape=jax.ShapeDtypeStruct((N, d_out), jnp.bfloat16),
    )(h1, agg2, dd, dd, Wr, b.reshape(1, d_out))


def _t3a_kernel(h_ref, w_ref, o_ref):
    h = h_ref[...].astype(_F32)
    o_ref[...] = jnp.dot(h, w_ref[...], preferred_element_type=_F32)


def _t3a(h2, Wl):
    k, d_out = Wl.shape
    return pl.pallas_call(
        _t3a_kernel,
        grid=(N // BM,),
        in_specs=[
            pl.BlockSpec((BM, k), lambda i: (i, 0)),
            pl.BlockSpec((k, d_out), lambda i: (0, 0)),
        ],
        out_specs=pl.BlockSpec((BM, d_out), lambda i: (i, 0)),
        out_shape=jax.ShapeDtypeStruct((N, d_out), _F32),
    )(h2, Wl)


def _t3b_kernel(h_ref, p0_ref, p1_ref, d0_ref, d1_ref, w_ref, b_ref, o_ref):
    h = h_ref[...].astype(_F32)
    acc = jnp.dot(h, w_ref[...], preferred_element_type=_F32)
    acc += (p0_ref[0][:, :64] + p1_ref[0][:, :64]) * _rdeg(d0_ref[0], d1_ref[0])
    acc += b_ref[...]
    m = jnp.max(acc, axis=1, keepdims=True)
    sh = acc - m
    lse = jnp.log(jnp.sum(jnp.exp(sh), axis=1, keepdims=True))
    o_ref[...] = sh - lse


def _t3b(h2, pp, dd, Wr, b):
    k, d_out = Wr.shape
    return pl.pallas_call(
        _t3b_kernel,
        grid=(N // BM,),
        in_specs=[
            pl.BlockSpec((BM, k), lambda i: (i, 0)),
            pl.BlockSpec((1, BM, 128), lambda i: (0, i, 0)),
            pl.BlockSpec((1, BM, 128), lambda i: (1, i, 0)),
            pl.BlockSpec((1, BM, 1), lambda i: (0, i, 0)),
            pl.BlockSpec((1, BM, 1), lambda i: (1, i, 0)),
            pl.BlockSpec((k, d_out), lambda i: (0, 0)),
            pl.BlockSpec((1, d_out), lambda i: (0, 0)),
        ],
        out_specs=pl.BlockSpec((BM, d_out), lambda i: (i, 0)),
        out_shape=jax.ShapeDtypeStruct((N, d_out), _F32),
    )(h2, pp, pp, dd, dd, Wr, b.reshape(1, d_out))


@functools.lru_cache(maxsize=1)
def _sc_kernels():
    return _make_s1(), _make_s2(), _make_s3()


def kernel(x, edge_index, W1l, W1r, b1, W2l, W2r, b2, W3l, W3r, b3):
    src = edge_index[0]
    dst = edge_index[1]
    _s1, _s2, _s3 = _sc_kernels()

    pp, ddflat = _s1(x, src, dst)
    pp = pp.reshape(2, N, 128)
    dd = ddflat.reshape(2, N, 1)

    h1 = _t1(x, pp, dd, W1l, W1r, b1)

    m2 = _t2a(h1, W2l.astype(jnp.bfloat16))
    agg2 = _s2(m2.reshape(8 * N, 128), src, dst).reshape(8, N, 128)
    h2 = _t2b(h1, agg2, dd, W2r.astype(jnp.bfloat16), b2)

    W3lp = jnp.concatenate([W3l, jnp.zeros((W3l.shape[0], 64), _F32)], axis=1)
    m3 = _t3a(h2, W3lp)
    p3 = _s3(m3, src, dst).reshape(2, N, 128)
    return _t3b(h2, p3, dd, W3r, b3)
